# baseline (speedup 1.0000x reference)
import functools

import jax
import jax.numpy as jnp
from jax import lax
from jax.experimental import pallas as pl
from jax.experimental.pallas import tpu as pltpu
from jax.experimental.pallas import tpu_sc as plsc

D_MODEL = 1024
D_HIDDEN = 4096
TOPK = 128
BLOCK_T = 256
N_TOK = 2048

_INT_MIN = -2147483648


def _encode_body(
    x_ref, w_enc_ref, b_enc_ref, b_pre_ref, pre_ref, key_ref, mu_ref, std_ref
):
    xb = x_ref[...]
    mu = jnp.mean(xb, axis=-1, keepdims=True)
    xc = xb - mu
    var = jnp.sum(xc * xc, axis=-1, keepdims=True) / (D_MODEL - 1)
    std = jnp.sqrt(var)
    xn = xc / (std + 1e-5) - b_pre_ref[...]
    pre = (
        jnp.dot(xn, w_enc_ref[...], preferred_element_type=jnp.float32)
        + b_enc_ref[...]
    )
    pre_ref[...] = pre
    u = pre.view(jnp.int32)
    key = jnp.where(u >= 0, u, u ^ jnp.int32(0x7FFFFFFF))
    key_ref[...] = key ^ jnp.int32(_INT_MIN)
    mu_ref[...] = mu
    std_ref[...] = std


def _encode_call(x2d, w_enc, b_enc2d, b_pre2d):
    n_tok = x2d.shape[0]
    grid = (n_tok // BLOCK_T,)
    return pl.pallas_call(
        _encode_body,
        grid=grid,
        in_specs=[
            pl.BlockSpec((BLOCK_T, D_MODEL), lambda i: (i, 0)),
            pl.BlockSpec((D_MODEL, D_HIDDEN), lambda i: (0, 0)),
            pl.BlockSpec((1, D_HIDDEN), lambda i: (0, 0)),
            pl.BlockSpec((1, D_MODEL), lambda i: (0, 0)),
        ],
        out_specs=[
            pl.BlockSpec((BLOCK_T, D_HIDDEN), lambda i: (i, 0)),
            pl.BlockSpec((BLOCK_T, D_HIDDEN), lambda i: (i, 0)),
            pl.BlockSpec((BLOCK_T, 1), lambda i: (i, 0)),
            pl.BlockSpec((BLOCK_T, 1), lambda i: (i, 0)),
        ],
        out_shape=[
            jax.ShapeDtypeStruct((n_tok, D_HIDDEN), jnp.float32),
            jax.ShapeDtypeStruct((n_tok, D_HIDDEN), jnp.int32),
            jax.ShapeDtypeStruct((n_tok, 1), jnp.float32),
            jax.ShapeDtypeStruct((n_tok, 1), jnp.float32),
        ],
    )(x2d, w_enc, b_enc2d, b_pre2d)


_INFO = plsc.get_sparse_core_info()
_NC = _INFO.num_cores
_NS = _INFO.num_subcores
_NW = _NC * _NS
_L = 16

_ROW_CHUNKS = D_HIDDEN // _L


def _lanes_i32():
    return lax.iota(jnp.int32, _L)


def _splat(x):
    return jnp.full((_L,), x, dtype=jnp.int32)


def _suffix_find(hist_ref, r_s):
    lanes = _lanes_i32()
    chunks = [hist_ref[pl.ds(g * _L, _L)] for g in range(16)]
    tot = jnp.zeros((_L,), jnp.int32)
    for g in range(16):
        tot = tot + jnp.where(lanes == g, jnp.sum(chunks[g]), 0)
    suftot = lax.rev(jnp.cumsum(lax.rev(tot, (0,))), (0,))
    carry_excl = suftot - tot

    nge_acc = jnp.zeros((_L,), jnp.int32)
    a_acc = jnp.zeros((_L,), jnp.int32)
    for g in range(16):
        h = chunks[g]
        ws = lax.rev(jnp.cumsum(lax.rev(h, (0,))), (0,))
        suf = ws + carry_excl[g]
        ge = suf >= r_s
        nge_acc = nge_acc + ge.astype(jnp.int32)
        a_acc = a_acc + jnp.where(ge, 0, h)
    nge = jnp.sum(nge_acc)
    d_star = nge - 1
    return d_star, jnp.sum(a_acc)


_UNROLL = 8


def _make_select_body(tpw):
  def _select_body(
    key_hbm, thr_hbm, key_a, key_b, cand_a, cand_b, hist_v, out_v, sem_a,
    sem_b
  ):
    wid = lax.axis_index("s") * _NC + lax.axis_index("c")
    base = wid * tpw
    last = base + tpw - 1

    def process(key_v, j):
        for g in range(16):
            hist_v[pl.ds(g * _L, _L)] = _splat(0)

        @plsc.parallel_loop(0, _ROW_CHUNKS, unroll=_UNROLL)
        def _(i):
            ub = key_v[pl.ds(i * _L, _L)]
            d = lax.shift_right_logical(ub, jnp.int32(24))
            plsc.addupdate_scatter(hist_v, [d], _splat(1))

        r1 = jnp.int32(TOPK)
        d1, a1 = _suffix_find(hist_v, r1)

        @plsc.parallel_loop(
            0, _ROW_CHUNKS, unroll=_UNROLL, carry=jnp.zeros((_L,), jnp.int32)
        )
        def c1v(i, off):
            ub = key_v[pl.ds(i * _L, _L)]
            d = lax.shift_right_logical(ub, jnp.int32(24))
            m = d == d1
            pos = jnp.cumsum(m.astype(jnp.int32))
            plsc.store_scatter(cand_a, [off + pos - 1], ub, mask=m)
            return off + plsc.all_reduce_population_count(m)

        c1 = c1v[0]

        def round_on(cand_src, cand_dst, c_in, r_in, shift):
            for g in range(16):
                hist_v[pl.ds(g * _L, _L)] = _splat(0)
            nchunks = (c_in + (_L - 1)) // _L

            @plsc.parallel_loop(0, nchunks, unroll=2)
            def _(i):
                ub = cand_src[pl.ds(i * _L, _L)]
                valid = (_lanes_i32() + i * _L) < c_in
                d = jnp.bitwise_and(
                    lax.shift_right_logical(ub, jnp.int32(shift)),
                    jnp.int32(0xFF),
                )
                plsc.addupdate_scatter(hist_v, [d], _splat(1), mask=valid)
            d_s, a_s = _suffix_find(hist_v, r_in)

            @plsc.parallel_loop(
                0, nchunks, unroll=2, carry=jnp.zeros((_L,), jnp.int32)
            )
            def c_outv(i, off):
                ub = cand_src[pl.ds(i * _L, _L)]
                valid = (_lanes_i32() + i * _L) < c_in
                d = jnp.bitwise_and(
                    lax.shift_right_logical(ub, jnp.int32(shift)),
                    jnp.int32(0xFF),
                )
                m = jnp.logical_and(d == d_s, valid)
                pos = jnp.cumsum(m.astype(jnp.int32))
                plsc.store_scatter(cand_dst, [off + pos - 1], ub, mask=m)
                return off + plsc.all_reduce_population_count(m)

            return c_outv[0], r_in - a_s

        c2, r2 = round_on(cand_a, cand_b, c1, r1 - a1, 16)
        c3, r3 = round_on(cand_b, cand_a, c2, r2, 8)
        _, _ = round_on(cand_a, cand_b, c3, r3, 0)

        ub_thr = plsc.load_gather(cand_b, [_splat(0)])
        key_thr = ub_thr ^ jnp.int32(_INT_MIN)
        plsc.store_scatter(
            out_v,
            [_splat(j)],
            key_thr,
            mask=_lanes_i32() == 0,
        )

    pltpu.async_copy(key_hbm.at[base], key_a, sem_a)

    def pair(p, _):
        t0 = base + 2 * p
        pltpu.async_copy(key_hbm.at[t0 + 1], key_b, sem_b)
        pltpu.make_async_copy(key_hbm.at[base], key_a, sem_a).wait()
        process(key_a, 2 * p)
        pltpu.async_copy(key_hbm.at[jnp.minimum(t0 + 2, last)], key_a, sem_a)
        pltpu.make_async_copy(key_hbm.at[base], key_b, sem_b).wait()
        process(key_b, 2 * p + 1)
        return 0

    lax.fori_loop(0, tpw // 2, pair, 0)
    pltpu.make_async_copy(key_hbm.at[base], key_a, sem_a).wait()

    pltpu.sync_copy(out_v, thr_hbm.at[pl.ds(base, tpw)])

  return _select_body


def _select_call(key):
    n_tok = key.shape[0]
    tpw = n_tok // _NW
    mesh = plsc.VectorSubcoreMesh(core_axis_name="c", subcore_axis_name="s")
    return pl.kernel(
        _make_select_body(tpw),
        mesh=mesh,
        compiler_params=pltpu.CompilerParams(needs_layout_passes=False),
        out_type=jax.ShapeDtypeStruct((n_tok,), jnp.int32),
        scratch_types=[
            pltpu.VMEM((D_HIDDEN,), jnp.int32),
            pltpu.VMEM((D_HIDDEN,), jnp.int32),
            pltpu.VMEM((D_HIDDEN,), jnp.int32),
            pltpu.VMEM((D_HIDDEN,), jnp.int32),
            pltpu.VMEM((256,), jnp.int32),
            pltpu.VMEM((tpw,), jnp.int32),
            pltpu.SemaphoreType.DMA,
            pltpu.SemaphoreType.DMA,
        ],
    )(key)


def _decode_body(pre_ref, thr_ref, w_dec_ref, b_pre_ref, mu_ref, std_ref, out_ref):
    pre = pre_ref[...]
    u = pre.view(jnp.int32)
    key = jnp.where(u >= 0, u, u ^ jnp.int32(0x7FFFFFFF))
    latents = jnp.where(key >= thr_ref[...], jax.nn.relu(pre), 0.0)
    recons = (
        jnp.dot(
            latents.astype(jnp.bfloat16),
            w_dec_ref[...].astype(jnp.bfloat16),
            preferred_element_type=jnp.float32,
        )
        + b_pre_ref[...]
    )
    out_ref[...] = recons * std_ref[...] + mu_ref[...]


def _decode_call(pre, thr2d, w_dec, b_pre2d, mu, std):
    n_tok = pre.shape[0]
    grid = (n_tok // BLOCK_T,)
    return pl.pallas_call(
        _decode_body,
        grid=grid,
        in_specs=[
            pl.BlockSpec((BLOCK_T, D_HIDDEN), lambda i: (i, 0)),
            pl.BlockSpec((BLOCK_T, 1), lambda i: (i, 0)),
            pl.BlockSpec((D_HIDDEN, D_MODEL), lambda i: (0, 0)),
            pl.BlockSpec((1, D_MODEL), lambda i: (0, 0)),
            pl.BlockSpec((BLOCK_T, 1), lambda i: (i, 0)),
            pl.BlockSpec((BLOCK_T, 1), lambda i: (i, 0)),
        ],
        out_specs=pl.BlockSpec((BLOCK_T, D_MODEL), lambda i: (i, 0)),
        out_shape=jax.ShapeDtypeStruct((n_tok, D_MODEL), jnp.float32),
    )(pre, thr2d, w_dec, b_pre2d, mu, std)


N_CHUNKS = 2


def kernel(x, w_enc, w_dec, b_enc, b_pre):
    b, t, d = x.shape
    x2d = x.reshape(b * t, d)
    b_enc2d = b_enc.reshape(1, D_HIDDEN)
    b_pre2d = b_pre.reshape(1, D_MODEL)
    ct = N_TOK // N_CHUNKS
    outs = []
    stages = [
        _encode_call(x2d[c * ct : (c + 1) * ct], w_enc, b_enc2d, b_pre2d)
        for c in range(N_CHUNKS)
    ]
    thrs = [_select_call(stages[c][1]) for c in range(N_CHUNKS)]
    for c in range(N_CHUNKS):
        pre, _, mu, std = stages[c]
        outs.append(
            _decode_call(
                pre, thrs[c].reshape(ct, 1), w_dec, b_pre2d, mu, std
            )
        )
    out = jnp.concatenate(outs, axis=0)
    return out.reshape(b, t, d)


# speedup vs baseline: 1.0811x; 1.0811x over previous
import functools

import jax
import jax.numpy as jnp
from jax import lax
from jax.experimental import pallas as pl
from jax.experimental.pallas import tpu as pltpu
from jax.experimental.pallas import tpu_sc as plsc

D_MODEL = 1024
D_HIDDEN = 4096
TOPK = 128
BLOCK_T = 256
N_TOK = 2048

_INT_MIN = -2147483648


def _encode_body(
    x_ref, w_enc_ref, b_enc_ref, b_pre_ref, pre_ref, key_ref, mu_ref, std_ref
):
    xb = x_ref[...]
    mu = jnp.mean(xb, axis=-1, keepdims=True)
    xc = xb - mu
    var = jnp.sum(xc * xc, axis=-1, keepdims=True) / (D_MODEL - 1)
    std = jnp.sqrt(var)
    xn = xc / (std + 1e-5) - b_pre_ref[...]
    pre = (
        jnp.dot(xn, w_enc_ref[...], preferred_element_type=jnp.float32)
        + b_enc_ref[...]
    )
    pre_ref[...] = pre
    u = pre.view(jnp.int32)
    key = jnp.where(u >= 0, u, u ^ jnp.int32(0x7FFFFFFF))
    key_ref[...] = key ^ jnp.int32(_INT_MIN)
    mu_ref[...] = mu
    std_ref[...] = std


def _encode_call(x2d, w_enc, b_enc2d, b_pre2d):
    n_tok = x2d.shape[0]
    grid = (n_tok // BLOCK_T,)
    return pl.pallas_call(
        _encode_body,
        grid=grid,
        in_specs=[
            pl.BlockSpec((BLOCK_T, D_MODEL), lambda i: (i, 0)),
            pl.BlockSpec((D_MODEL, D_HIDDEN), lambda i: (0, 0)),
            pl.BlockSpec((1, D_HIDDEN), lambda i: (0, 0)),
            pl.BlockSpec((1, D_MODEL), lambda i: (0, 0)),
        ],
        out_specs=[
            pl.BlockSpec((BLOCK_T, D_HIDDEN), lambda i: (i, 0)),
            pl.BlockSpec((BLOCK_T, D_HIDDEN), lambda i: (i, 0)),
            pl.BlockSpec((BLOCK_T, 1), lambda i: (i, 0)),
            pl.BlockSpec((BLOCK_T, 1), lambda i: (i, 0)),
        ],
        out_shape=[
            jax.ShapeDtypeStruct((n_tok, D_HIDDEN), jnp.float32),
            jax.ShapeDtypeStruct((n_tok, D_HIDDEN), jnp.int32),
            jax.ShapeDtypeStruct((n_tok, 1), jnp.float32),
            jax.ShapeDtypeStruct((n_tok, 1), jnp.float32),
        ],
    )(x2d, w_enc, b_enc2d, b_pre2d)


_INFO = plsc.get_sparse_core_info()
_NC = _INFO.num_cores
_NS = _INFO.num_subcores
_NW = _NC * _NS
_L = 16

_ROW_CHUNKS = D_HIDDEN // _L


def _lanes_i32():
    return lax.iota(jnp.int32, _L)


def _splat(x):
    return jnp.full((_L,), x, dtype=jnp.int32)


def _suffix_find(hist_ref, r_s):
    lanes = _lanes_i32()
    chunks = [hist_ref[pl.ds(g * _L, _L)] for g in range(16)]
    tot = jnp.zeros((_L,), jnp.int32)
    for g in range(16):
        tot = tot + jnp.where(lanes == g, jnp.sum(chunks[g]), 0)
    suftot = lax.rev(jnp.cumsum(lax.rev(tot, (0,))), (0,))
    carry_excl = suftot - tot

    nge_acc = jnp.zeros((_L,), jnp.int32)
    a_acc = jnp.zeros((_L,), jnp.int32)
    for g in range(16):
        h = chunks[g]
        ws = lax.rev(jnp.cumsum(lax.rev(h, (0,))), (0,))
        suf = ws + carry_excl[g]
        ge = suf >= r_s
        nge_acc = nge_acc + ge.astype(jnp.int32)
        a_acc = a_acc + jnp.where(ge, 0, h)
    nge = jnp.sum(nge_acc)
    d_star = nge - 1
    return d_star, jnp.sum(a_acc)


_UNROLL = 8


def _make_select_body(tpw):
  def _select_body(
    key_hbm, thr_hbm, key_a, key_b, cand_a, cand_b, hist_v, out_v, sem_a,
    sem_b
  ):
    wid = lax.axis_index("s") * _NC + lax.axis_index("c")
    base = wid * tpw
    last = base + tpw - 1

    def process(key_v, j):
        for g in range(16):
            hist_v[pl.ds(g * _L, _L)] = _splat(0)

        @plsc.parallel_loop(0, _ROW_CHUNKS, unroll=_UNROLL)
        def _(i):
            ub = key_v[pl.ds(i * _L, _L)]
            d = lax.shift_right_logical(ub, jnp.int32(24))
            plsc.addupdate_scatter(hist_v, [d], _splat(1))

        r1 = jnp.int32(TOPK)
        d1, a1 = _suffix_find(hist_v, r1)

        @plsc.parallel_loop(
            0, _ROW_CHUNKS, unroll=_UNROLL, carry=jnp.zeros((_L,), jnp.int32)
        )
        def c1v(i, off):
            ub = key_v[pl.ds(i * _L, _L)]
            d = lax.shift_right_logical(ub, jnp.int32(24))
            m = d == d1
            pos = jnp.cumsum(m.astype(jnp.int32))
            plsc.store_scatter(cand_a, [off + pos - 1], ub, mask=m)
            return off + plsc.all_reduce_population_count(m)

        c1 = c1v[0]

        def round_on(cand_src, cand_dst, c_in, r_in, shift):
            for g in range(16):
                hist_v[pl.ds(g * _L, _L)] = _splat(0)
            nchunks = (c_in + (_L - 1)) // _L

            @plsc.parallel_loop(0, nchunks, unroll=2)
            def _(i):
                ub = cand_src[pl.ds(i * _L, _L)]
                valid = (_lanes_i32() + i * _L) < c_in
                d = jnp.bitwise_and(
                    lax.shift_right_logical(ub, jnp.int32(shift)),
                    jnp.int32(0xFF),
                )
                plsc.addupdate_scatter(hist_v, [d], _splat(1), mask=valid)
            d_s, a_s = _suffix_find(hist_v, r_in)

            @plsc.parallel_loop(
                0, nchunks, unroll=2, carry=jnp.zeros((_L,), jnp.int32)
            )
            def c_outv(i, off):
                ub = cand_src[pl.ds(i * _L, _L)]
                valid = (_lanes_i32() + i * _L) < c_in
                d = jnp.bitwise_and(
                    lax.shift_right_logical(ub, jnp.int32(shift)),
                    jnp.int32(0xFF),
                )
                m = jnp.logical_and(d == d_s, valid)
                pos = jnp.cumsum(m.astype(jnp.int32))
                plsc.store_scatter(cand_dst, [off + pos - 1], ub, mask=m)
                return off + plsc.all_reduce_population_count(m)

            return c_outv[0], r_in - a_s

        c2, r2 = round_on(cand_a, cand_b, c1, r1 - a1, 16)
        c3, r3 = round_on(cand_b, cand_a, c2, r2, 8)
        _, _ = round_on(cand_a, cand_b, c3, r3, 0)

        ub_thr = plsc.load_gather(cand_b, [_splat(0)])
        key_thr = ub_thr ^ jnp.int32(_INT_MIN)
        plsc.store_scatter(
            out_v,
            [_splat(j)],
            key_thr,
            mask=_lanes_i32() == 0,
        )

    pltpu.async_copy(key_hbm.at[base], key_a, sem_a)

    def pair(p, _):
        t0 = base + 2 * p
        pltpu.async_copy(key_hbm.at[t0 + 1], key_b, sem_b)
        pltpu.make_async_copy(key_hbm.at[base], key_a, sem_a).wait()
        process(key_a, 2 * p)
        pltpu.async_copy(key_hbm.at[jnp.minimum(t0 + 2, last)], key_a, sem_a)
        pltpu.make_async_copy(key_hbm.at[base], key_b, sem_b).wait()
        process(key_b, 2 * p + 1)
        return 0

    lax.fori_loop(0, tpw // 2, pair, 0)
    pltpu.make_async_copy(key_hbm.at[base], key_a, sem_a).wait()

    pltpu.sync_copy(out_v, thr_hbm.at[pl.ds(base, tpw)])

  return _select_body


def _select_call(key):
    n_tok = key.shape[0]
    tpw = n_tok // _NW
    mesh = plsc.VectorSubcoreMesh(core_axis_name="c", subcore_axis_name="s")
    return pl.kernel(
        _make_select_body(tpw),
        mesh=mesh,
        compiler_params=pltpu.CompilerParams(needs_layout_passes=False),
        out_type=jax.ShapeDtypeStruct((n_tok,), jnp.int32),
        scratch_types=[
            pltpu.VMEM((D_HIDDEN,), jnp.int32),
            pltpu.VMEM((D_HIDDEN,), jnp.int32),
            pltpu.VMEM((D_HIDDEN,), jnp.int32),
            pltpu.VMEM((D_HIDDEN,), jnp.int32),
            pltpu.VMEM((256,), jnp.int32),
            pltpu.VMEM((tpw,), jnp.int32),
            pltpu.SemaphoreType.DMA,
            pltpu.SemaphoreType.DMA,
        ],
    )(key)


def _decode_body(pre_ref, thr_ref, w_dec_ref, b_pre_ref, mu_ref, std_ref, out_ref):
    pre = pre_ref[...]
    u = pre.view(jnp.int32)
    key = jnp.where(u >= 0, u, u ^ jnp.int32(0x7FFFFFFF))
    latents = jnp.where(key >= thr_ref[...], jax.nn.relu(pre), 0.0)
    recons = (
        jnp.dot(
            latents.astype(jnp.bfloat16),
            w_dec_ref[...].astype(jnp.bfloat16),
            preferred_element_type=jnp.float32,
        )
        + b_pre_ref[...]
    )
    out_ref[...] = recons * std_ref[...] + mu_ref[...]


def _decode_call(pre, thr2d, w_dec, b_pre2d, mu, std):
    n_tok = pre.shape[0]
    grid = (n_tok // BLOCK_T,)
    return pl.pallas_call(
        _decode_body,
        grid=grid,
        in_specs=[
            pl.BlockSpec((BLOCK_T, D_HIDDEN), lambda i: (i, 0)),
            pl.BlockSpec((BLOCK_T, 1), lambda i: (i, 0)),
            pl.BlockSpec((D_HIDDEN, D_MODEL), lambda i: (0, 0)),
            pl.BlockSpec((1, D_MODEL), lambda i: (0, 0)),
            pl.BlockSpec((BLOCK_T, 1), lambda i: (i, 0)),
            pl.BlockSpec((BLOCK_T, 1), lambda i: (i, 0)),
        ],
        out_specs=pl.BlockSpec((BLOCK_T, D_MODEL), lambda i: (i, 0)),
        out_shape=jax.ShapeDtypeStruct((n_tok, D_MODEL), jnp.float32),
    )(pre, thr2d, w_dec, b_pre2d, mu, std)


N_CHUNKS = 1


def kernel(x, w_enc, w_dec, b_enc, b_pre):
    b, t, d = x.shape
    x2d = x.reshape(b * t, d)
    b_enc2d = b_enc.reshape(1, D_HIDDEN)
    b_pre2d = b_pre.reshape(1, D_MODEL)
    ct = N_TOK // N_CHUNKS
    outs = []
    stages = [
        _encode_call(x2d[c * ct : (c + 1) * ct], w_enc, b_enc2d, b_pre2d)
        for c in range(N_CHUNKS)
    ]
    thrs = [_select_call(stages[c][1]) for c in range(N_CHUNKS)]
    for c in range(N_CHUNKS):
        pre, _, mu, std = stages[c]
        outs.append(
            _decode_call(
                pre, thrs[c].reshape(ct, 1), w_dec, b_pre2d, mu, std
            )
        )
    out = jnp.concatenate(outs, axis=0)
    return out.reshape(b, t, d)


# speedup vs baseline: 1.0978x; 1.0155x over previous
import functools

import jax
import jax.numpy as jnp
from jax import lax
from jax.experimental import pallas as pl
from jax.experimental.pallas import tpu as pltpu
from jax.experimental.pallas import tpu_sc as plsc

D_MODEL = 1024
D_HIDDEN = 4096
TOPK = 128
BLOCK_T = 256
N_TOK = 2048

_INT_MIN = -2147483648


def _encode_body(
    x_ref, w_enc_ref, b_enc_ref, b_pre_ref, key_ref, mu_ref, std_ref
):
    xb = x_ref[...]
    mu = jnp.mean(xb, axis=-1, keepdims=True)
    xc = xb - mu
    var = jnp.sum(xc * xc, axis=-1, keepdims=True) / (D_MODEL - 1)
    std = jnp.sqrt(var)
    xn = xc / (std + 1e-5) - b_pre_ref[...]
    pre = (
        jnp.dot(xn, w_enc_ref[...], preferred_element_type=jnp.float32)
        + b_enc_ref[...]
    )
    u = pre.view(jnp.int32)
    key = jnp.where(u >= 0, u, u ^ jnp.int32(0x7FFFFFFF))
    key_ref[...] = key ^ jnp.int32(_INT_MIN)
    mu_ref[...] = mu
    std_ref[...] = std


def _encode_call(x2d, w_enc, b_enc2d, b_pre2d):
    n_tok = x2d.shape[0]
    grid = (n_tok // BLOCK_T,)
    return pl.pallas_call(
        _encode_body,
        grid=grid,
        in_specs=[
            pl.BlockSpec((BLOCK_T, D_MODEL), lambda i: (i, 0)),
            pl.BlockSpec((D_MODEL, D_HIDDEN), lambda i: (0, 0)),
            pl.BlockSpec((1, D_HIDDEN), lambda i: (0, 0)),
            pl.BlockSpec((1, D_MODEL), lambda i: (0, 0)),
        ],
        out_specs=[
            pl.BlockSpec((BLOCK_T, D_HIDDEN), lambda i: (i, 0)),
            pl.BlockSpec((BLOCK_T, 1), lambda i: (i, 0)),
            pl.BlockSpec((BLOCK_T, 1), lambda i: (i, 0)),
        ],
        out_shape=[
            jax.ShapeDtypeStruct((n_tok, D_HIDDEN), jnp.int32),
            jax.ShapeDtypeStruct((n_tok, 1), jnp.float32),
            jax.ShapeDtypeStruct((n_tok, 1), jnp.float32),
        ],
    )(x2d, w_enc, b_enc2d, b_pre2d)


_INFO = plsc.get_sparse_core_info()
_NC = _INFO.num_cores
_NS = _INFO.num_subcores
_NW = _NC * _NS
_L = 16

_ROW_CHUNKS = D_HIDDEN // _L


def _lanes_i32():
    return lax.iota(jnp.int32, _L)


def _splat(x):
    return jnp.full((_L,), x, dtype=jnp.int32)


def _suffix_find(hist_ref, r_s):
    lanes = _lanes_i32()
    chunks = [hist_ref[pl.ds(g * _L, _L)] for g in range(16)]
    tot = jnp.zeros((_L,), jnp.int32)
    for g in range(16):
        tot = tot + jnp.where(lanes == g, jnp.sum(chunks[g]), 0)
    suftot = lax.rev(jnp.cumsum(lax.rev(tot, (0,))), (0,))
    carry_excl = suftot - tot

    nge_acc = jnp.zeros((_L,), jnp.int32)
    a_acc = jnp.zeros((_L,), jnp.int32)
    for g in range(16):
        h = chunks[g]
        ws = lax.rev(jnp.cumsum(lax.rev(h, (0,))), (0,))
        suf = ws + carry_excl[g]
        ge = suf >= r_s
        nge_acc = nge_acc + ge.astype(jnp.int32)
        a_acc = a_acc + jnp.where(ge, 0, h)
    nge = jnp.sum(nge_acc)
    d_star = nge - 1
    return d_star, jnp.sum(a_acc)


_UNROLL = 8


def _make_select_body(tpw):
  def _select_body(
    key_hbm, thr_hbm, key_a, key_b, cand_a, cand_b, hist_v, out_v, sem_a,
    sem_b
  ):
    wid = lax.axis_index("s") * _NC + lax.axis_index("c")
    base = wid * tpw
    last = base + tpw - 1

    def process(key_v, j):
        for g in range(16):
            hist_v[pl.ds(g * _L, _L)] = _splat(0)

        @plsc.parallel_loop(0, _ROW_CHUNKS, unroll=_UNROLL)
        def _(i):
            ub = key_v[pl.ds(i * _L, _L)]
            d = lax.shift_right_logical(ub, jnp.int32(24))
            plsc.addupdate_scatter(hist_v, [d], _splat(1))

        r1 = jnp.int32(TOPK)
        d1, a1 = _suffix_find(hist_v, r1)

        @plsc.parallel_loop(
            0, _ROW_CHUNKS, unroll=_UNROLL, carry=jnp.zeros((_L,), jnp.int32)
        )
        def c1v(i, off):
            ub = key_v[pl.ds(i * _L, _L)]
            d = lax.shift_right_logical(ub, jnp.int32(24))
            m = d == d1
            pos = jnp.cumsum(m.astype(jnp.int32))
            plsc.store_scatter(cand_a, [off + pos - 1], ub, mask=m)
            return off + plsc.all_reduce_population_count(m)

        c1 = c1v[0]

        def round_on(cand_src, cand_dst, c_in, r_in, shift):
            for g in range(16):
                hist_v[pl.ds(g * _L, _L)] = _splat(0)
            nchunks = (c_in + (_L - 1)) // _L

            @plsc.parallel_loop(0, nchunks, unroll=2)
            def _(i):
                ub = cand_src[pl.ds(i * _L, _L)]
                valid = (_lanes_i32() + i * _L) < c_in
                d = jnp.bitwise_and(
                    lax.shift_right_logical(ub, jnp.int32(shift)),
                    jnp.int32(0xFF),
                )
                plsc.addupdate_scatter(hist_v, [d], _splat(1), mask=valid)
            d_s, a_s = _suffix_find(hist_v, r_in)

            @plsc.parallel_loop(
                0, nchunks, unroll=2, carry=jnp.zeros((_L,), jnp.int32)
            )
            def c_outv(i, off):
                ub = cand_src[pl.ds(i * _L, _L)]
                valid = (_lanes_i32() + i * _L) < c_in
                d = jnp.bitwise_and(
                    lax.shift_right_logical(ub, jnp.int32(shift)),
                    jnp.int32(0xFF),
                )
                m = jnp.logical_and(d == d_s, valid)
                pos = jnp.cumsum(m.astype(jnp.int32))
                plsc.store_scatter(cand_dst, [off + pos - 1], ub, mask=m)
                return off + plsc.all_reduce_population_count(m)

            return c_outv[0], r_in - a_s

        c2, r2 = round_on(cand_a, cand_b, c1, r1 - a1, 16)
        c3, r3 = round_on(cand_b, cand_a, c2, r2, 8)
        _, _ = round_on(cand_a, cand_b, c3, r3, 0)

        ub_thr = plsc.load_gather(cand_b, [_splat(0)])
        key_thr = ub_thr ^ jnp.int32(_INT_MIN)
        plsc.store_scatter(
            out_v,
            [_splat(j)],
            key_thr,
            mask=_lanes_i32() == 0,
        )

    pltpu.async_copy(key_hbm.at[base], key_a, sem_a)

    def pair(p, _):
        t0 = base + 2 * p
        pltpu.async_copy(key_hbm.at[t0 + 1], key_b, sem_b)
        pltpu.make_async_copy(key_hbm.at[base], key_a, sem_a).wait()
        process(key_a, 2 * p)
        pltpu.async_copy(key_hbm.at[jnp.minimum(t0 + 2, last)], key_a, sem_a)
        pltpu.make_async_copy(key_hbm.at[base], key_b, sem_b).wait()
        process(key_b, 2 * p + 1)
        return 0

    lax.fori_loop(0, tpw // 2, pair, 0)
    pltpu.make_async_copy(key_hbm.at[base], key_a, sem_a).wait()

    pltpu.sync_copy(out_v, thr_hbm.at[pl.ds(base, tpw)])

  return _select_body


def _select_call(key):
    n_tok = key.shape[0]
    tpw = n_tok // _NW
    mesh = plsc.VectorSubcoreMesh(core_axis_name="c", subcore_axis_name="s")
    return pl.kernel(
        _make_select_body(tpw),
        mesh=mesh,
        compiler_params=pltpu.CompilerParams(needs_layout_passes=False),
        out_type=jax.ShapeDtypeStruct((n_tok,), jnp.int32),
        scratch_types=[
            pltpu.VMEM((D_HIDDEN,), jnp.int32),
            pltpu.VMEM((D_HIDDEN,), jnp.int32),
            pltpu.VMEM((D_HIDDEN,), jnp.int32),
            pltpu.VMEM((D_HIDDEN,), jnp.int32),
            pltpu.VMEM((256,), jnp.int32),
            pltpu.VMEM((tpw,), jnp.int32),
            pltpu.SemaphoreType.DMA,
            pltpu.SemaphoreType.DMA,
        ],
    )(key)


def _decode_body(kb_ref, thr_ref, w_dec_ref, b_pre_ref, mu_ref, std_ref, out_ref):
    key = kb_ref[...] ^ jnp.int32(_INT_MIN)
    u = jnp.where(key >= 0, key, key ^ jnp.int32(0x7FFFFFFF))
    pre = u.view(jnp.float32)
    latents = jnp.where(key >= thr_ref[...], jax.nn.relu(pre), 0.0)
    recons = (
        jnp.dot(
            latents.astype(jnp.bfloat16),
            w_dec_ref[...].astype(jnp.bfloat16),
            preferred_element_type=jnp.float32,
        )
        + b_pre_ref[...]
    )
    out_ref[...] = recons * std_ref[...] + mu_ref[...]


def _decode_call(kb, thr2d, w_dec, b_pre2d, mu, std):
    n_tok = kb.shape[0]
    grid = (n_tok // BLOCK_T,)
    return pl.pallas_call(
        _decode_body,
        grid=grid,
        in_specs=[
            pl.BlockSpec((BLOCK_T, D_HIDDEN), lambda i: (i, 0)),
            pl.BlockSpec((BLOCK_T, 1), lambda i: (i, 0)),
            pl.BlockSpec((D_HIDDEN, D_MODEL), lambda i: (0, 0)),
            pl.BlockSpec((1, D_MODEL), lambda i: (0, 0)),
            pl.BlockSpec((BLOCK_T, 1), lambda i: (i, 0)),
            pl.BlockSpec((BLOCK_T, 1), lambda i: (i, 0)),
        ],
        out_specs=pl.BlockSpec((BLOCK_T, D_MODEL), lambda i: (i, 0)),
        out_shape=jax.ShapeDtypeStruct((n_tok, D_MODEL), jnp.float32),
    )(kb, thr2d, w_dec, b_pre2d, mu, std)


N_CHUNKS = 1


def kernel(x, w_enc, w_dec, b_enc, b_pre):
    b, t, d = x.shape
    x2d = x.reshape(b * t, d)
    b_enc2d = b_enc.reshape(1, D_HIDDEN)
    b_pre2d = b_pre.reshape(1, D_MODEL)
    ct = N_TOK // N_CHUNKS
    outs = []
    stages = [
        _encode_call(x2d[c * ct : (c + 1) * ct], w_enc, b_enc2d, b_pre2d)
        for c in range(N_CHUNKS)
    ]
    thrs = [_select_call(stages[c][0]) for c in range(N_CHUNKS)]
    for c in range(N_CHUNKS):
        kb, mu, std = stages[c]
        outs.append(
            _decode_call(
                kb, thrs[c].reshape(ct, 1), w_dec, b_pre2d, mu, std
            )
        )
    out = jnp.concatenate(outs, axis=0)
    return out.reshape(b, t, d)
